# baseline (speedup 1.0000x reference)
import functools

import jax
import jax.numpy as jnp
from jax import lax
from jax.experimental import pallas as pl
from jax.experimental.pallas import tpu as pltpu
from jax.experimental.pallas import tpu_sc as plsc

B = 16384
N_NUM = 13
N_CAT = 26
VOCAB = 100000
D = 32
N_TOK = 1 + N_NUM + N_CAT
D_OUT = N_TOK * D
CAT_COL0 = (1 + N_NUM) * D
NUM_W = 512

NC, NS = 2, 16
NW = NC * NS
CHUNK = 1024
ITEMS_PER_FIELD = B // CHUNK
N_ITEMS = N_CAT * ITEMS_PER_FIELD
ITEMS_PER_W = N_ITEMS // NW
SUB = 128
NSUB = CHUNK // SUB


def _sc_gather(table_flat, idx4):
    mesh = plsc.VectorSubcoreMesh(core_axis_name="c", subcore_axis_name="s")

    @functools.partial(
        pl.kernel,
        out_type=jax.ShapeDtypeStruct((B, D_OUT), jnp.float32),
        mesh=mesh,
        scratch_types=[
            pltpu.VMEM((NSUB, SUB), jnp.int32),
            pltpu.VMEM((CHUNK, D), jnp.float32),
            pltpu.SemaphoreType.DMA,
        ],
        compiler_params=pltpu.CompilerParams(use_tc_tiling_on_sc=False),
    )
    def k(table_hbm, idx_hbm, out_hbm, idx_v, rows_v, gsem):
        wid = lax.axis_index("s") * NC + lax.axis_index("c")

        @pl.loop(0, ITEMS_PER_W)
        def _(it):
            item = wid * ITEMS_PER_W + it
            j = item // ITEMS_PER_FIELD
            c = item - j * ITEMS_PER_FIELD
            pltpu.sync_copy(idx_hbm.at[j, c], idx_v)
            cps = [
                pltpu.async_copy(
                    table_hbm.at[idx_v.at[s]],
                    rows_v.at[pl.ds(s * SUB, SUB)],
                    gsem,
                )
                for s in range(NSUB)
            ]
            for cp in cps:
                cp.wait()
            pltpu.sync_copy(
                rows_v,
                out_hbm.at[pl.ds(c * CHUNK, CHUNK), pl.ds(CAT_COL0 + j * D, D)],
            )

    return k(table_flat, idx4)


def _num_body(xp_ref, s_ref, w2_ref, b2_ref, g_ref, out_ref):
    xsel = jnp.dot(xp_ref[...], s_ref[...], preferred_element_type=jnp.float32)
    num = xsel * w2_ref[...] + b2_ref[...]
    col = lax.broadcasted_iota(jnp.int32, out_ref.shape, 1)
    out_ref[...] = jnp.where(col < CAT_COL0, num, g_ref[...])


def _num_fill(xp, sel, w2, b2, g):
    bb = 2048
    grid = (B // bb,)
    return pl.pallas_call(
        _num_body,
        grid=grid,
        in_specs=[
            pl.BlockSpec((bb, 16), lambda i: (i, 0)),
            pl.BlockSpec((16, NUM_W), lambda i: (0, 0)),
            pl.BlockSpec((1, NUM_W), lambda i: (0, 0)),
            pl.BlockSpec((1, NUM_W), lambda i: (0, 0)),
            pl.BlockSpec((bb, NUM_W), lambda i: (i, 0)),
        ],
        out_specs=pl.BlockSpec((bb, NUM_W), lambda i: (i, 0)),
        out_shape=jax.ShapeDtypeStruct((B, D_OUT), jnp.float32),
        input_output_aliases={4: 0},
        compiler_params=pltpu.CompilerParams(
            dimension_semantics=("arbitrary",),
        ),
    )(xp, sel, w2, b2, g)


def kernel(X_num, X_cat, num_weight, num_bias, cat_tables):
    xc = X_cat.astype(jnp.int32)
    offs = (jnp.arange(N_CAT, dtype=jnp.int32) * VOCAB)[None, :]
    idx4 = (xc + offs).T.reshape(N_CAT, ITEMS_PER_FIELD, NSUB, SUB)
    table_flat = cat_tables.reshape(N_CAT * VOCAB, D)
    g = _sc_gather(table_flat, idx4)

    xp = jnp.concatenate(
        [jnp.zeros((B, 1), jnp.float32), X_num, jnp.zeros((B, 2), jnp.float32)],
        axis=1,
    )
    tok_of_col = jnp.arange(NUM_W, dtype=jnp.int32) // D
    sel = (tok_of_col[None, :] == jnp.arange(16, dtype=jnp.int32)[:, None]).astype(
        jnp.float32
    )
    pad = NUM_W - CAT_COL0
    w2 = jnp.concatenate(
        [jnp.zeros((D,), jnp.float32), num_weight.reshape(-1), jnp.zeros((pad,), jnp.float32)]
    )[None, :]
    b2 = jnp.concatenate(
        [jnp.zeros((D,), jnp.float32), num_bias.reshape(-1), jnp.zeros((pad,), jnp.float32)]
    )[None, :]
    return _num_fill(xp, sel, w2, b2, g)
